# baseline (speedup 1.0000x reference)
import functools

import jax
import jax.numpy as jnp
from jax import lax
from jax.experimental import pallas as pl
from jax.experimental.pallas import tpu as pltpu
from jax.experimental.pallas import tpu_sc as plsc

B = 16384
D = 64
NC = 2
NS = 16
NW = NC * NS
BPW = B // NW
CHUNK = 128
NCH = BPW // CHUNK
NT = 3

_mesh = plsc.VectorSubcoreMesh(core_axis_name="c", subcore_axis_name="s")


@functools.partial(
    pl.kernel,
    mesh=_mesh,
    compiler_params=pltpu.CompilerParams(use_tc_tiling_on_sc=False),
    out_type=jax.ShapeDtypeStruct((NT * B, D), jnp.float32),
    scratch_types=[
        pltpu.VMEM((NT * NCH, CHUNK), jnp.int32),
        pltpu.VMEM((NT * NCH, CHUNK, D), jnp.float32),
        pltpu.SemaphoreType.DMA,
    ],
)
def _gather3(src_hbm, dst_hbm, rels_hbm, node_hbm, rel_hbm, out_hbm,
             idx_v, rows_v, sem):
    wid = lax.axis_index("s") * NC + lax.axis_index("c")
    base = wid * BPW
    idx_srcs = (src_hbm, dst_hbm, rels_hbm)
    tables = (node_hbm, node_hbm, rel_hbm)

    for t in range(NT):
        for c in range(NCH):
            j = t * NCH + c
            pltpu.sync_copy(idx_srcs[t].at[pl.ds(base + c * CHUNK, CHUNK)],
                            idx_v.at[j])
    copies = []
    for t in range(NT):
        for c in range(NCH):
            j = t * NCH + c
            copies.append(
                pltpu.async_copy(tables[t].at[idx_v.at[j]], rows_v.at[j], sem))
    for cp in copies:
        cp.wait()
    for t in range(NT):
        for c in range(NCH):
            j = t * NCH + c
            pltpu.sync_copy(
                rows_v.at[j],
                out_hbm.at[pl.ds(t * B + base + c * CHUNK, CHUNK)])


def kernel(src, dst, rels, node_table, rel_table):
    out = _gather3(src.astype(jnp.int32), dst.astype(jnp.int32),
                   rels.astype(jnp.int32), node_table, rel_table)
    return out.reshape(NT, B, D)


# speedup vs baseline: 2.4331x; 2.4331x over previous
import functools

import jax
import jax.numpy as jnp
from jax import lax
from jax.experimental import pallas as pl
from jax.experimental.pallas import tpu as pltpu
from jax.experimental.pallas import tpu_sc as plsc

B = 16384
D = 64
NC = 2
NS = 16
NW = NC * NS
BPW = B // NW
NT = 3
L = 16
NG = BPW // L

_mesh = plsc.VectorSubcoreMesh(core_axis_name="c", subcore_axis_name="s")


@functools.partial(
    pl.kernel,
    mesh=_mesh,
    out_type=jax.ShapeDtypeStruct((NT * B, D), jnp.float32),
    scratch_types=[
        pltpu.VMEM((NT * BPW,), jnp.int32),
        pltpu.VMEM((BPW, D), jnp.float32),
        pltpu.SemaphoreType.DMA,
    ],
)
def _gather3(src_hbm, dst_hbm, rels_hbm, node_hbm, rel_hbm, out_hbm,
             idx_v, rows_v, sem):
    wid = lax.axis_index("s") * NC + lax.axis_index("c")
    base = wid * BPW
    idx_srcs = (src_hbm, dst_hbm, rels_hbm)
    tables = (node_hbm, node_hbm, rel_hbm)

    for t in range(NT):
        pltpu.sync_copy(idx_srcs[t].at[pl.ds(base, BPW)],
                        idx_v.at[pl.ds(t * BPW, BPW)])

    for t in range(NT):
        table = tables[t]

        def group_body(g, _, table=table, t=t):
            vec = idx_v[pl.ds(t * BPW + g * L, L)]
            for j in range(L):
                i = vec[j]
                tid = lax.shift_right_logical(i, 3)
                r = i & 7
                pltpu.async_copy(table.at[tid, r], rows_v.at[g * L + j], sem)
            return ()

        lax.fori_loop(0, NG, group_body, (), unroll=False)

        def drain_body(g, _, table=table):
            for j in range(L):
                pltpu.make_async_copy(table.at[0, 0],
                                      rows_v.at[g * L + j], sem).wait()
            return ()

        lax.fori_loop(0, NG, drain_body, (), unroll=False)

        pltpu.sync_copy(rows_v, out_hbm.at[pl.ds(t * B + base, BPW)])


def kernel(src, dst, rels, node_table, rel_table):
    node3 = node_table.reshape(125000, 8, D)
    rel3 = rel_table.reshape(125, 8, D)
    out = _gather3(src.astype(jnp.int32), dst.astype(jnp.int32),
                   rels.astype(jnp.int32), node3, rel3)
    return out.reshape(NT, B, D)
